# baseline (speedup 1.0000x reference)
import jax
import jax.numpy as jnp
from jax.experimental import pallas as pl
from jax.experimental.pallas import tpu as pltpu

D = 768
NUM_HEADS = 12
HEAD_DIM = D // NUM_HEADS
ATT_SCALE = HEAD_DIM ** (-0.5)
FF = 3072
FF_TILE = 1536
N_LAYERS = 12
LN_EPS = 1e-5
BB = 16
NCHUNK = 2
S = 80


def _norm(x):
    mu = jnp.mean(x, axis=-1, keepdims=True)
    var = jnp.mean(jnp.square(x - mu), axis=-1, keepdims=True)
    return (x - mu) * jax.lax.rsqrt(var + LN_EPS)


def _gelu_tanh(x):
    c = 0.7978845608028654
    c2 = c * 0.044715
    v = x * (c + c2 * (x * x))
    h = 0.5 * x
    return h + h * jnp.tanh(v)


def _attn_chunk(x, mask2d, w_in, b_in, wo_p, bo):
    rows = x.shape[0]
    cb = rows // S

    xn = _norm(x).astype(jnp.bfloat16)
    ab = (jnp.dot(xn, w_in, preferred_element_type=jnp.float32) + b_in)
    ab = ab.astype(jnp.bfloat16).reshape(cb, S, 2 * NUM_HEADS * 128)

    ctx_heads = []
    for h in range(NUM_HEADS):
        a_h = ab[:, :, 128 * h:128 * h + 128]
        b_h = ab[:, :, 1536 + 128 * h:1536 + 128 * h + 128]
        sc = jax.lax.dot_general(a_h, b_h, (((2,), (2,)), ((0,), (0,))),
                                 preferred_element_type=jnp.float32)
        e = jnp.exp(sc + mask2d[None])
        p = (e / jnp.sum(e, axis=-1, keepdims=True)).astype(jnp.bfloat16)
        ctx_heads.append(jax.lax.dot_general(
            p, b_h, (((2,), (1,)), ((0,), (0,))),
            preferred_element_type=jnp.float32).astype(jnp.bfloat16))
    ctx = jnp.concatenate(ctx_heads, axis=-1)
    ctx = ctx.reshape(rows, NUM_HEADS * 128)

    attn = jnp.dot(ctx, wo_p, preferred_element_type=jnp.float32) + bo
    return x + attn


def _mlp_chunk(x, w1, b1, w2, b2):
    xn2 = _norm(x).astype(jnp.bfloat16)
    acc = x + b2
    for t in range(FF // FF_TILE):
        fo = t * FF_TILE
        ht = jnp.dot(xn2, w1[:, fo:fo + FF_TILE],
                     preferred_element_type=jnp.float32) + b1[:, fo:fo + FF_TILE]
        ht = _gelu_tanh(ht).astype(jnp.bfloat16)
        acc = acc + jnp.dot(ht, w2[fo:fo + FF_TILE, :],
                            preferred_element_type=jnp.float32)
    return acc


def _encoder_kernel(x_hbm, mask_ref, w_in_ref, b_in_ref, wo_ref, bo_ref,
                    w1_ref, b1_ref, w2_ref, b2_ref, out_ref, dma_sem):
    layer = pl.program_id(1)

    @pl.when(layer == 0)
    def _():
        nb = pl.program_id(0)
        cp = pltpu.make_async_copy(x_hbm.at[pl.ds(nb * BB, BB)], out_ref,
                                   dma_sem)
        cp.start()
        cp.wait()

    mask2d = mask_ref[0, 0]
    cb = BB // NCHUNK
    xs = []
    for c in range(NCHUNK):
        x = out_ref[c * cb:(c + 1) * cb].reshape(cb * S, D)
        xs.append(_attn_chunk(x, mask2d, w_in_ref[0], b_in_ref[0],
                              wo_ref[0], bo_ref[0]))
    for c in range(NCHUNK):
        y = _mlp_chunk(xs[c], w1_ref[0], b1_ref[0], w2_ref[0], b2_ref[0])
        out_ref[c * cb:(c + 1) * cb] = y.reshape(cb, S, D)


def kernel(hidden, mask, ln1_g, ln1_b, qkv_w, qkv_b, wo, bo,
           ln2_g, ln2_b, w1, b1, w2, b2):
    B, S_, _ = hidden.shape
    nb = B // BB
    L = N_LAYERS

    w_f = qkv_w * ln1_g[:, 0, :, None]
    b_f = qkv_b[:, 0, :] + jnp.einsum('ld,ldo->lo', ln1_b[:, 0, :], qkv_w)
    wq = (w_f[:, :, 0:D] * ATT_SCALE).reshape(L, D, NUM_HEADS, HEAD_DIM)
    wk = w_f[:, :, D:2 * D].reshape(L, D, NUM_HEADS, HEAD_DIM)
    wv = w_f[:, :, 2 * D:3 * D].reshape(L, D, NUM_HEADS, HEAD_DIM)
    bq = (b_f[:, 0:D] * ATT_SCALE).reshape(L, NUM_HEADS, HEAD_DIM)
    bk = b_f[:, D:2 * D].reshape(L, NUM_HEADS, HEAD_DIM)
    bv = b_f[:, 2 * D:3 * D].reshape(L, NUM_HEADS, HEAD_DIM)

    a_w = jnp.concatenate([wq, jnp.zeros_like(wq)], axis=-1)
    b_w = jnp.concatenate([wk, wv], axis=-1)
    w_in = jnp.concatenate([a_w, b_w], axis=2).reshape(L, D, 2 * NUM_HEADS * 128)
    a_b = jnp.concatenate([bq, jnp.zeros_like(bq)], axis=-1)
    b_b = jnp.concatenate([bk, bv], axis=-1)
    b_in = jnp.concatenate([a_b, b_b], axis=1).reshape(L, 1, 2 * NUM_HEADS * 128)

    wo_r = wo.reshape(L, NUM_HEADS, HEAD_DIM, D)
    wo_p = jnp.concatenate([jnp.zeros_like(wo_r), wo_r],
                           axis=2).reshape(L, NUM_HEADS * 128, D)

    w1_f = w1 * ln2_g[:, 0, :, None]
    b1_f = (b1[:, 0, :] + jnp.einsum('ld,ldo->lo', ln2_b[:, 0, :], w1))
    b1_f = b1_f.reshape(L, 1, FF)

    w_in = w_in.astype(jnp.bfloat16)
    wo_p = wo_p.astype(jnp.bfloat16)
    w1_f = w1_f.astype(jnp.bfloat16)
    w2 = w2.astype(jnp.bfloat16)

    return pl.pallas_call(
        _encoder_kernel,
        out_shape=jax.ShapeDtypeStruct((B, S_, D), jnp.float32),
        grid_spec=pltpu.PrefetchScalarGridSpec(
            num_scalar_prefetch=0,
            grid=(nb, N_LAYERS),
            in_specs=[
                pl.BlockSpec(memory_space=pl.ANY),
                pl.BlockSpec((1, 1, S_, S_), lambda b, l: (0, 0, 0, 0)),
                pl.BlockSpec((1, D, 2 * NUM_HEADS * 128),
                             lambda b, l: (l, 0, 0)),
                pl.BlockSpec((1, 1, 2 * NUM_HEADS * 128),
                             lambda b, l: (l, 0, 0)),
                pl.BlockSpec((1, NUM_HEADS * 128, D),
                             lambda b, l: (l, 0, 0)),
                pl.BlockSpec((1, 1, D), lambda b, l: (l, 0, 0)),
                pl.BlockSpec((1, D, FF), lambda b, l: (l, 0, 0)),
                pl.BlockSpec((1, 1, FF), lambda b, l: (l, 0, 0)),
                pl.BlockSpec((1, FF, D), lambda b, l: (l, 0, 0)),
                pl.BlockSpec((1, 1, D), lambda b, l: (l, 0, 0)),
            ],
            out_specs=pl.BlockSpec((BB, S_, D), lambda b, l: (b, 0, 0)),
            scratch_shapes=[pltpu.SemaphoreType.DMA],
        ),
        compiler_params=pltpu.CompilerParams(
            dimension_semantics=("parallel", "arbitrary"),
            vmem_limit_bytes=56 * 1024 * 1024,
        ),
    )(hidden, mask, w_in, b_in, wo_p, bo, w1_f, b1_f, w2, b2)
